# Initial kernel scaffold; baseline (speedup 1.0000x reference)
import jax
import jax.numpy as jnp
from jax.experimental import pallas as pl


def kernel(pred_points, gt_points, gt_normals, epoch):
    raise NotImplementedError("write your pallas kernel here")


# speedup vs baseline: 1.5488x; 1.5488x over previous
import math

import jax
import jax.numpy as jnp
from jax.experimental import pallas as pl
from jax.experimental.pallas import tpu as pltpu

W_CHAMFER = 1.0
W_CHAMFER_OPP = 1.0
W_EMD = 0.1
SINKHORN_EPS = 0.05
SINKHORN_ITERS = 10


def _pcr_body(gt_ref, pr_ref, s1_ref, s2_ref, emd_ref):
    gt = gt_ref[0]
    pr = pr_ref[0]
    n = gt.shape[0]
    m = pr.shape[0]

    ab = jax.lax.dot_general(
        gt, pr,
        dimension_numbers=(((1,), (1,)), ((), ())),
        preferred_element_type=jnp.float32,
    )
    aa = jnp.sum(gt * gt, axis=1, keepdims=True)
    bb = jnp.sum(pr * pr, axis=1, keepdims=True)
    bb_row = bb.reshape((1, m))
    C = jnp.maximum(aa + bb_row - 2.0 * ab, 0.0)

    s1 = jnp.sum(jnp.min(C, axis=1))
    s2 = jnp.sum(jnp.min(C, axis=0))

    inv_eps = 1.0 / SINKHORN_EPS
    K = C * (-inv_eps)
    loga = -math.log(float(m))
    logb = -math.log(float(n))
    F = jnp.zeros((1, m), dtype=jnp.float32)
    G = jnp.zeros((n, 1), dtype=jnp.float32)
    for _ in range(SINKHORN_ITERS):
        T = G + K
        mx = jnp.max(T, axis=0, keepdims=True)
        F = loga - (jnp.log(jnp.sum(jnp.exp(T - mx), axis=0, keepdims=True)) + mx)
        T2 = F + K
        mx2 = jnp.max(T2, axis=1, keepdims=True)
        G = logb - (jnp.log(jnp.sum(jnp.exp(T2 - mx2), axis=1, keepdims=True)) + mx2)

    logP = F + G + K + (loga + logb)
    emd = jnp.sum(jnp.exp(logP) * C)

    s1_ref[...] = jnp.full((1, 1, 128), s1, dtype=jnp.float32)
    s2_ref[...] = jnp.full((1, 1, 128), s2, dtype=jnp.float32)
    emd_ref[...] = jnp.full((1, 1, 128), emd, dtype=jnp.float32)


@jax.jit
def _pcr_loss(pred_points, gt_points):
    B, N, D = gt_points.shape
    M = pred_points.shape[1]
    out_sds = jax.ShapeDtypeStruct((B, 1, 128), jnp.float32)
    s1, s2, emd = pl.pallas_call(
        _pcr_body,
        grid=(B,),
        in_specs=[
            pl.BlockSpec((1, N, D), lambda b: (b, 0, 0)),
            pl.BlockSpec((1, M, D), lambda b: (b, 0, 0)),
        ],
        out_specs=[
            pl.BlockSpec((1, 1, 128), lambda b: (b, 0, 0)),
            pl.BlockSpec((1, 1, 128), lambda b: (b, 0, 0)),
            pl.BlockSpec((1, 1, 128), lambda b: (b, 0, 0)),
        ],
        out_shape=[out_sds, out_sds, out_sds],
        compiler_params=pltpu.CompilerParams(
            dimension_semantics=("parallel",),
        ),
    )(gt_points, pred_points)
    chamfer_loss = (jnp.sum(s1[:, 0, 0]) / (B * N)
                    + W_CHAMFER_OPP * jnp.sum(s2[:, 0, 0]) / (B * M))
    emd_loss = jnp.mean(emd[:, 0, 0])
    loss = W_CHAMFER * chamfer_loss + W_EMD * emd_loss
    return loss, chamfer_loss, emd_loss


def kernel(pred_points, gt_points, gt_normals, epoch):
    del gt_normals, epoch
    return _pcr_loss(pred_points.astype(jnp.float32),
                     gt_points.astype(jnp.float32))


# speedup vs baseline: 3.0680x; 1.9808x over previous
import math

import jax
import jax.numpy as jnp
from jax.experimental import pallas as pl
from jax.experimental.pallas import tpu as pltpu

W_CHAMFER = 1.0
W_CHAMFER_OPP = 1.0
W_EMD = 0.1
SINKHORN_EPS = 0.05
SINKHORN_ITERS = 10
_FLOOR = 1e-37


def _pcr_body(gt_ref, pr_ref, s1_ref, s2_ref, emd_ref):
    gt = gt_ref[0]
    pr = pr_ref[0]
    n = gt.shape[0]
    m = pr.shape[0]

    ab = jax.lax.dot_general(
        gt, pr,
        dimension_numbers=(((1,), (1,)), ((), ())),
        preferred_element_type=jnp.float32,
    )
    aa = jnp.sum(gt * gt, axis=1, keepdims=True)
    bb = jnp.sum(pr * pr, axis=1, keepdims=True)
    bb_row = bb.reshape((1, m))
    C = jnp.maximum(aa + bb_row - 2.0 * ab, 0.0)

    alpha = jnp.min(C, axis=1, keepdims=True)
    colmin = jnp.min(C, axis=0, keepdims=True)
    beta = jnp.min(C - alpha, axis=0, keepdims=True)
    s1_sum = jnp.sum(alpha)
    s2_sum = jnp.sum(colmin)

    inv_eps = 1.0 / SINKHORN_EPS
    a = 1.0 / float(m)
    b = 1.0 / float(n)

    E = jnp.exp((alpha + beta - C) * inv_eps)

    s = jnp.sum(jnp.exp((beta - C) * inv_eps), axis=0, keepdims=True)
    v = a / jnp.maximum(s, _FLOOR)
    t = jnp.sum(E * v, axis=1, keepdims=True)
    u = b / jnp.maximum(t, _FLOOR)
    for _ in range(SINKHORN_ITERS - 1):
        s = jnp.sum(u * E, axis=0, keepdims=True)
        v = a / jnp.maximum(s, _FLOOR)
        t = jnp.sum(E * v, axis=1, keepdims=True)
        u = b / jnp.maximum(t, _FLOOR)

    t2 = jnp.sum(E * C * v, axis=1, keepdims=True)
    emd = (a * b) * jnp.sum(u * t2)

    s1_ref[...] = jnp.full((1, 1, 128), s1_sum, dtype=jnp.float32)
    s2_ref[...] = jnp.full((1, 1, 128), s2_sum, dtype=jnp.float32)
    emd_ref[...] = jnp.full((1, 1, 128), emd, dtype=jnp.float32)


@jax.jit
def _pcr_loss(pred_points, gt_points):
    B, N, D = gt_points.shape
    M = pred_points.shape[1]
    out_sds = jax.ShapeDtypeStruct((B, 1, 128), jnp.float32)
    s1, s2, emd = pl.pallas_call(
        _pcr_body,
        grid=(B,),
        in_specs=[
            pl.BlockSpec((1, N, D), lambda b: (b, 0, 0)),
            pl.BlockSpec((1, M, D), lambda b: (b, 0, 0)),
        ],
        out_specs=[
            pl.BlockSpec((1, 1, 128), lambda b: (b, 0, 0)),
            pl.BlockSpec((1, 1, 128), lambda b: (b, 0, 0)),
            pl.BlockSpec((1, 1, 128), lambda b: (b, 0, 0)),
        ],
        out_shape=[out_sds, out_sds, out_sds],
        compiler_params=pltpu.CompilerParams(
            dimension_semantics=("parallel",),
        ),
    )(gt_points, pred_points)
    chamfer_loss = (jnp.sum(s1[:, 0, 0]) / (B * N)
                    + W_CHAMFER_OPP * jnp.sum(s2[:, 0, 0]) / (B * M))
    emd_loss = jnp.mean(emd[:, 0, 0])
    loss = W_CHAMFER * chamfer_loss + W_EMD * emd_loss
    return loss, chamfer_loss, emd_loss


def kernel(pred_points, gt_points, gt_normals, epoch):
    del gt_normals, epoch
    return _pcr_loss(pred_points.astype(jnp.float32),
                     gt_points.astype(jnp.float32))
